# Initial kernel scaffold; baseline (speedup 1.0000x reference)
import jax
import jax.numpy as jnp
from jax.experimental import pallas as pl


def kernel(x, edge_index, W1_l, b1, W1_r, W2_l, b2, W2_r, W3_l, b3, W3_r):
    raise NotImplementedError("write your pallas kernel here")


# speedup vs baseline: 2.1464x; 2.1464x over previous
import functools

import jax
import jax.numpy as jnp
from jax import lax
from jax.experimental import pallas as pl
from jax.experimental.pallas import tpu as pltpu
from jax.experimental.pallas import tpu_sc as plsc

N_NODES = 10000
D = 256
N_EDGES = 160000

NC = 2
NS = 16
NW = NC * NS
H = 10240
ZST = H // NS
B = 128
EPT = 5120
E_PAD = NW * EPT
NBLK = EPT // B


def _sc_segsum_body(src_hbm, dst_hbm, x_hbm, slab_hbm,
                    srcbuf, dstraw, dstloc, rows, zbuf, sem):
    cid = lax.axis_index("c")
    sid = lax.axis_index("s")

    zv = jnp.zeros((16,), jnp.float32)

    @pl.loop(0, B)
    def _(r):
        @pl.loop(0, D, step=16)
        def _(j):
            zbuf[r, pl.ds(j, 16)] = zv

    zbase = cid * H + sid * ZST

    @pl.loop(0, ZST // B)
    def _(i):
        pltpu.sync_copy(zbuf, slab_hbm.at[pl.ds(zbase + i * B, B)])

    plsc.subcore_barrier()

    wid = sid * NC + cid
    ebase = wid * EPT
    soff = cid * H

    @pl.loop(0, NBLK)
    def _(i):
        off = ebase + i * B
        pltpu.sync_copy(src_hbm.at[pl.ds(off, B)], srcbuf)
        pltpu.sync_copy(dst_hbm.at[pl.ds(off, B)], dstraw)

        @pl.loop(0, B, step=16)
        def _(j):
            dstloc[pl.ds(j, 16)] = dstraw[pl.ds(j, 16)] + soff

        pltpu.sync_copy(x_hbm.at[srcbuf], rows)
        pltpu.async_copy(rows, slab_hbm.at[dstloc], sem, add=True).wait()


@jax.jit
def _sc_segsum(src_p, dst_p, x):
    mesh = plsc.VectorSubcoreMesh(core_axis_name="c", subcore_axis_name="s")
    f = pl.kernel(
        _sc_segsum_body,
        out_type=jax.ShapeDtypeStruct((NC * H, D), jnp.float32),
        mesh=mesh,
        scratch_types=[
            pltpu.VMEM((B,), jnp.int32),
            pltpu.VMEM((B,), jnp.int32),
            pltpu.VMEM((B,), jnp.int32),
            pltpu.VMEM((B, D), jnp.float32),
            pltpu.VMEM((B, D), jnp.float32),
            pltpu.SemaphoreType.DMA,
        ],
    )
    return f(src_p, dst_p, x)


def _dense_body(apply_relu, s0_ref, s1_ref, x_ref, wl_ref, wr_ref, b_ref,
                o_ref):
    agg = s0_ref[...] + s1_ref[...]
    h = jnp.dot(agg, wl_ref[...], preferred_element_type=jnp.float32)
    h = h + jnp.dot(x_ref[...], wr_ref[...], preferred_element_type=jnp.float32)
    h = h + b_ref[...]
    nrm = jnp.sqrt(jnp.sum(h * h, axis=1, keepdims=True))
    h = h / jnp.maximum(nrm, 1e-12)
    if apply_relu:
        h = jnp.maximum(h, 0.0)
    o_ref[...] = h


def _dense(slab, x, wlT, wrT, b2d, apply_relu):
    R = 80
    grid = (N_NODES // R,)
    off1 = H // R
    return pl.pallas_call(
        functools.partial(_dense_body, apply_relu),
        grid=grid,
        in_specs=[
            pl.BlockSpec((R, D), lambda i: (i, 0)),
            pl.BlockSpec((R, D), lambda i, o=off1: (i + o, 0)),
            pl.BlockSpec((R, D), lambda i: (i, 0)),
            pl.BlockSpec((D, D), lambda i: (0, 0)),
            pl.BlockSpec((D, D), lambda i: (0, 0)),
            pl.BlockSpec((1, D), lambda i: (0, 0)),
        ],
        out_specs=pl.BlockSpec((R, D), lambda i: (i, 0)),
        out_shape=jax.ShapeDtypeStruct((N_NODES, D), jnp.float32),
    )(slab, slab, x, wlT, wrT, b2d)


def kernel(x, edge_index, W1_l, b1, W1_r, W2_l, b2, W2_r, W3_l, b3, W3_r):
    src = edge_index[0]
    dst = edge_index[1]
    pad = E_PAD - N_EDGES
    src_p = jnp.concatenate([src, jnp.zeros((pad,), jnp.int32)])
    dst_p = jnp.concatenate([dst, jnp.full((pad,), N_NODES, jnp.int32)])

    slab1 = _sc_segsum(src_p, dst_p, x)
    h1 = _dense(slab1, x, W1_l.T, W1_r.T, b1.reshape(1, D), True)
    slab2 = _sc_segsum(src_p, dst_p, h1)
    h2 = _dense(slab2, h1, W2_l.T, W2_r.T, b2.reshape(1, D), True)
    slab3 = _sc_segsum(src_p, dst_p, h2)
    out = _dense(slab3, h2, W3_l.T, W3_r.T, b3.reshape(1, D), False)
    return out
